# Initial kernel scaffold; baseline (speedup 1.0000x reference)
import jax
import jax.numpy as jnp
from jax.experimental import pallas as pl


def kernel(x, y):
    raise NotImplementedError("write your pallas kernel here")


# speedup vs baseline: 1.0014x; 1.0014x over previous
import jax
import jax.numpy as jnp
from jax.experimental import pallas as pl


def _loss_kernel(x_ref, y_ref, o_ref):
    x = x_ref[...]
    y = y_ref[...]
    cols = jax.lax.broadcasted_iota(jnp.int32, x.shape, 1)
    mask = cols == y
    o_ref[...] = jnp.where(mask, -jnp.log(x), 0.0)


def kernel(x, y):
    y = y.astype(jnp.int32)
    return pl.pallas_call(
        _loss_kernel,
        out_shape=jax.ShapeDtypeStruct(x.shape, x.dtype),
    )(x, y)
